# baseline (speedup 1.0000x reference)
import functools

import jax
import jax.numpy as jnp
from jax import lax
from jax.experimental import pallas as pl
from jax.experimental.pallas import tpu as pltpu
from jax.experimental.pallas import tpu_sc as plsc

_NUM_CODES = 512
_LATENT = 64
_E = 2304
_NB = 4
_EB = _E // _NB
_LANES = 16
_GROUPS = _E // _LANES
_TROW = 528


def _build_tables(W):
    C, K = W.shape
    iota = jnp.broadcast_to(jnp.arange(K, dtype=jnp.int32), (C, K))
    ws, order = lax.sort((W, iota), dimension=1, num_keys=1, is_stable=True)
    is_new = jnp.concatenate(
        [jnp.ones((C, 1), bool), ws[:, 1:] != ws[:, :-1]], axis=1)
    tagged = jnp.where(is_new, (iota << 9) | order, 0)
    fidx = lax.cummax(tagged, axis=1) & (2 ** 9 - 1)
    expw, expnw = jnp.exp(ws), jnp.exp(-ws)
    kio = jnp.arange(K, dtype=jnp.int32)
    jio = jnp.arange(_TROW, dtype=jnp.int32)
    t_pre = (kio[:, None] < jio[None, :]).astype(jnp.float32)
    t_suf = (kio[:, None] >= jio[None, :]).astype(jnp.float32)
    e_pre = jnp.stack([expw, expw * ws], axis=1)
    e_suf = jnp.stack([expnw, expnw * ws], axis=1)
    hi = jax.lax.Precision.HIGHEST
    pre = jnp.einsum("cik,kj->cij", e_pre, t_pre, precision=hi)
    suf = jnp.einsum("cik,kj->cij", e_suf, t_suf, precision=hi)

    z1 = jnp.zeros((C, 1), jnp.float32)
    inf = jnp.full((C, 1), jnp.inf, jnp.float32)
    zi = jnp.zeros((C, 1), jnp.int32)
    pad = jnp.zeros((C, _TROW - (K + 1)), jnp.float32)
    rows = [
        jnp.concatenate([-inf, ws, pad], axis=1),
        jnp.concatenate([ws, inf, pad], axis=1),
        pre[:, 0],
        pre[:, 1],
        suf[:, 0],
        suf[:, 1],
        jax.lax.bitcast_convert_type(
            jnp.concatenate([zi, fidx, pad.astype(jnp.int32)], axis=1),
            jnp.float32),
        jax.lax.bitcast_convert_type(
            jnp.concatenate([fidx, zi, pad.astype(jnp.int32)], axis=1),
            jnp.float32),
    ]
    return jnp.concatenate(rows, axis=1)


def _make_sc_call(C):
    mesh = plsc.VectorSubcoreMesh(core_axis_name="c", subcore_axis_name="s")
    n_workers = 32
    chans_per_worker = C // n_workers

    @functools.partial(
        pl.kernel,
        mesh=mesh,
        out_type=[
            jax.ShapeDtypeStruct((C, _E), jnp.float32),
            jax.ShapeDtypeStruct((C, _E), jnp.float32),
            jax.ShapeDtypeStruct((C, _E), jnp.int32),
        ],
        scratch_types=(
            [pltpu.VMEM((_E,), jnp.float32)] * 2
            + [pltpu.VMEM((8 * _TROW,), jnp.float32)] * 2
            + [pltpu.VMEM((_E,), jnp.float32)] * 4
            + [pltpu.VMEM((_E,), jnp.int32)] * 2
            + [pltpu.SemaphoreType.DMA] * 3
        ),
        compiler_params=pltpu.CompilerParams(needs_layout_passes=False),
    )
    def sc_quantize(x_hbm, t_hbm, soft_hbm, hard_hbm, idx_hbm,
                    x0, x1, t0, t1, s0, s1, h0, h1, i0, i1,
                    sin0, sin1, sout):
        wid = lax.axis_index("s") * 2 + lax.axis_index("c")
        c0 = wid * chans_per_worker

        bufs = [(x0, t0, s0, h0, i0), (x1, t1, s1, h1, i1)]
        sins = [sin0, sin1]
        dins = []
        for t in range(chans_per_worker):
            dins.append((
                pltpu.async_copy(x_hbm.at[c0 + t], bufs[t][0], sins[t]),
                pltpu.async_copy(t_hbm.at[c0 + t], bufs[t][1], sins[t]),
            ))

        douts = []
        for t in range(chans_per_worker):
            for d in dins[t]:
                d.wait()
            xv, tv, sv, hv, iv = bufs[t]

            @plsc.parallel_loop(0, _GROUPS, unroll=24)
            def group(i):
                base = i * _LANES
                vv = xv[pl.ds(base, _LANES)]
                j = jnp.zeros((_LANES,), jnp.int32)
                for step in (256, 128, 64, 32, 16, 8, 4, 2, 1):
                    probe = plsc.load_gather(tv, [j + (_TROW + step - 1)])
                    j = jnp.where(probe < vv, j + step, j)
                probe = plsc.load_gather(tv, [j + _TROW])
                j = jnp.where(probe < vv, j + 1, j)

                wl = plsc.load_gather(tv, [j])
                wr = plsc.load_gather(tv, [j + _TROW])
                a = plsc.load_gather(tv, [j + 2 * _TROW])
                aw = plsc.load_gather(tv, [j + 3 * _TROW])
                b = plsc.load_gather(tv, [j + 4 * _TROW])
                bw = plsc.load_gather(tv, [j + 5 * _TROW])
                fl = plsc.bitcast(
                    plsc.load_gather(tv, [j + 6 * _TROW]), jnp.int32)
                fr = plsc.bitcast(
                    plsc.load_gather(tv, [j + 7 * _TROW]), jnp.int32)

                u = jnp.exp(2.0 * vv)
                soft = (aw + u * bw) / (a + u * b)
                dl = vv - wl
                dr = wr - vv
                pick_l = (dl < dr) | ((dl == dr) & (fl < fr))
                sv[pl.ds(base, _LANES)] = soft
                hv[pl.ds(base, _LANES)] = jnp.where(pick_l, wl, wr)
                iv[pl.ds(base, _LANES)] = jnp.where(pick_l, fl, fr)

            c = c0 + t
            douts.append(pltpu.async_copy(sv, soft_hbm.at[c], sout))
            douts.append(pltpu.async_copy(hv, hard_hbm.at[c], sout))
            douts.append(pltpu.async_copy(iv, idx_hbm.at[c], sout))

        for d in douts:
            d.wait()

    return sc_quantize


def kernel(z, W):
    B, C, H, Wd = z.shape
    X = jnp.transpose(z, (1, 0, 2, 3)).reshape(C, _E)
    T = _build_tables(W)
    soft, hard, idx = _make_sc_call(C)(X, T)

    def back(a):
        return jnp.transpose(a.reshape(C, B, H, Wd), (1, 2, 3, 0))

    return (back(soft), back(hard), back(idx))


# speedup vs baseline: 1.0232x; 1.0232x over previous
import functools

import jax
import jax.numpy as jnp
from jax import lax
from jax.experimental import pallas as pl
from jax.experimental.pallas import tpu as pltpu
from jax.experimental.pallas import tpu_sc as plsc

_NUM_CODES = 512
_LATENT = 64
_E = 2304
_NB = 4
_EB = _E // _NB
_LANES = 16
_GROUPS = _E // _LANES
_TROW = 528


def _build_tables(W):
    C, K = W.shape
    iota = jnp.broadcast_to(jnp.arange(K, dtype=jnp.int32), (C, K))
    ws, order = lax.sort((W, iota), dimension=1, num_keys=1, is_stable=True)
    is_new = jnp.concatenate(
        [jnp.ones((C, 1), bool), ws[:, 1:] != ws[:, :-1]], axis=1)
    tagged = jnp.where(is_new, (iota << 9) | order, 0)
    fidx = lax.cummax(tagged, axis=1) & (2 ** 9 - 1)
    expw, expnw = jnp.exp(ws), jnp.exp(-ws)
    kio = jnp.arange(K, dtype=jnp.int32)
    jio = jnp.arange(_TROW, dtype=jnp.int32)
    t_pre = (kio[:, None] < jio[None, :]).astype(jnp.float32)
    t_suf = (kio[:, None] >= jio[None, :]).astype(jnp.float32)
    e_pre = jnp.stack([expw, expw * ws], axis=1)
    e_suf = jnp.stack([expnw, expnw * ws], axis=1)
    hi = jax.lax.Precision.HIGHEST
    pre = jnp.einsum("cik,kj->cij", e_pre, t_pre, precision=hi)
    suf = jnp.einsum("cik,kj->cij", e_suf, t_suf, precision=hi)

    z1 = jnp.zeros((C, 1), jnp.float32)
    inf = jnp.full((C, 1), jnp.inf, jnp.float32)
    zi = jnp.zeros((C, 1), jnp.int32)
    pad = jnp.zeros((C, _TROW - (K + 1)), jnp.float32)
    rows = [
        jnp.concatenate([-inf, ws, pad], axis=1),
        jnp.concatenate([ws, inf, pad], axis=1),
        pre[:, 0],
        pre[:, 1],
        suf[:, 0],
        suf[:, 1],
        jax.lax.bitcast_convert_type(
            jnp.concatenate([zi, fidx, pad.astype(jnp.int32)], axis=1),
            jnp.float32),
        jax.lax.bitcast_convert_type(
            jnp.concatenate([fidx, zi, pad.astype(jnp.int32)], axis=1),
            jnp.float32),
    ]
    return jnp.concatenate(rows, axis=1)


def _make_sc_call(C):
    mesh = plsc.VectorSubcoreMesh(core_axis_name="c", subcore_axis_name="s")
    n_workers = 32
    chans_per_worker = C // n_workers

    @functools.partial(
        pl.kernel,
        mesh=mesh,
        out_type=[
            jax.ShapeDtypeStruct((C, _E), jnp.float32),
            jax.ShapeDtypeStruct((C, _E), jnp.float32),
            jax.ShapeDtypeStruct((C, _E), jnp.int32),
        ],
        scratch_types=(
            [pltpu.VMEM((_E,), jnp.float32)] * 2
            + [pltpu.VMEM((8 * _TROW,), jnp.float32)] * 2
            + [pltpu.VMEM((_E,), jnp.float32)] * 4
            + [pltpu.VMEM((_E,), jnp.int32)] * 2
            + [pltpu.SemaphoreType.DMA] * 3
        ),
        compiler_params=pltpu.CompilerParams(needs_layout_passes=False),
    )
    def sc_quantize(x_hbm, t_hbm, soft_hbm, hard_hbm, idx_hbm,
                    x0, x1, t0, t1, s0, s1, h0, h1, i0, i1,
                    sin0, sin1, sout):
        wid = lax.axis_index("s") * 2 + lax.axis_index("c")
        c0 = wid * chans_per_worker

        bufs = [(x0, t0, s0, h0, i0), (x1, t1, s1, h1, i1)]
        sins = [sin0, sin1]
        dins = []
        for t in range(chans_per_worker):
            dins.append((
                pltpu.async_copy(x_hbm.at[c0 + t], bufs[t][0], sins[t]),
                pltpu.async_copy(t_hbm.at[c0 + t], bufs[t][1], sins[t]),
            ))

        douts = []
        for t in range(chans_per_worker):
            for d in dins[t]:
                d.wait()
            xv, tv, sv, hv, iv = bufs[t]

            @plsc.parallel_loop(0, _GROUPS, unroll=16)
            def group(i):
                base = i * _LANES
                vv = xv[pl.ds(base, _LANES)]
                j = jnp.zeros((_LANES,), jnp.int32)
                for step in (256, 128, 64, 32, 16, 8, 4, 2, 1):
                    probe = plsc.load_gather(tv, [j + (_TROW + step - 1)])
                    j = jnp.where(probe < vv, j + step, j)
                probe = plsc.load_gather(tv, [j + _TROW])
                j = jnp.where(probe < vv, j + 1, j)

                wl = plsc.load_gather(tv, [j])
                wr = plsc.load_gather(tv, [j + _TROW])
                a = plsc.load_gather(tv, [j + 2 * _TROW])
                aw = plsc.load_gather(tv, [j + 3 * _TROW])
                b = plsc.load_gather(tv, [j + 4 * _TROW])
                bw = plsc.load_gather(tv, [j + 5 * _TROW])
                fl = plsc.bitcast(
                    plsc.load_gather(tv, [j + 6 * _TROW]), jnp.int32)
                fr = plsc.bitcast(
                    plsc.load_gather(tv, [j + 7 * _TROW]), jnp.int32)

                u = jnp.exp(2.0 * vv)
                soft = (aw + u * bw) / (a + u * b)
                dl = vv - wl
                dr = wr - vv
                pick_l = (dl < dr) | ((dl == dr) & (fl < fr))
                sv[pl.ds(base, _LANES)] = soft
                hv[pl.ds(base, _LANES)] = jnp.where(pick_l, wl, wr)
                iv[pl.ds(base, _LANES)] = jnp.where(pick_l, fl, fr)

            c = c0 + t
            douts.append(pltpu.async_copy(sv, soft_hbm.at[c], sout))
            douts.append(pltpu.async_copy(hv, hard_hbm.at[c], sout))
            douts.append(pltpu.async_copy(iv, idx_hbm.at[c], sout))

        for d in douts:
            d.wait()

    return sc_quantize


def kernel(z, W):
    B, C, H, Wd = z.shape
    X = jnp.transpose(z, (1, 0, 2, 3)).reshape(C, _E)
    T = _build_tables(W)
    soft, hard, idx = _make_sc_call(C)(X, T)

    def back(a):
        return jnp.transpose(a.reshape(C, B, H, Wd), (1, 2, 3, 0))

    return (back(soft), back(hard), back(idx))
